# baseline (speedup 1.0000x reference)
import functools

import jax
import jax.numpy as jnp
from jax import lax
from jax.experimental import pallas as pl
from jax.experimental.pallas import tpu as pltpu
from jax.experimental.pallas import tpu_sc as plsc

_NUM_CLASSES = 1000000
_OUT_DIM = 64
_BATCH = 16384
_L = 16
_NC = 2
_NS = 16
_NW = _NC * _NS
_K = 2

_mesh = plsc.VectorSubcoreMesh(core_axis_name="c", subcore_axis_name="s")


def _make_embed(bk):
    b_per_w = bk // _NW
    ng = b_per_w // _L

    @functools.partial(
        pl.kernel,
        mesh=_mesh,
        out_type=jax.ShapeDtypeStruct((bk, _OUT_DIM), jnp.float32),
        scratch_types=[
            pltpu.VMEM((b_per_w,), jnp.int32),
            pltpu.VMEM((b_per_w,), jnp.int32),
            pltpu.VMEM((_L,), jnp.int32),
            pltpu.VMEM((b_per_w, _OUT_DIM), jnp.float32),
            pltpu.SemaphoreType.DMA,
            pltpu.SemaphoreType.DMA,
        ],
    )
    def _embed(labels_hbm, train_hbm, drop_hbm, table_hbm, out_hbm,
               idx_v, drop_v, train_v, rows_v, sem, sem2):
        wid = lax.axis_index("s") * _NC + lax.axis_index("c")
        base = wid * b_per_w
        pltpu.sync_copy(labels_hbm.at[pl.ds(base, b_per_w)], idx_v)
        pltpu.sync_copy(drop_hbm.at[pl.ds(base, b_per_w)], drop_v)
        pltpu.sync_copy(train_hbm, train_v)
        trn = train_v[...]
        null_row = jnp.full((_L,), _NUM_CLASSES, dtype=jnp.int32)
        for i in range(ng):
            sl = pl.ds(i * _L, _L)
            idx_v[sl] = jnp.where((trn != 0) & (drop_v[sl] != 0),
                                  null_row, idx_v[sl])

        @pl.loop(0, ng)
        def _(g):
            lab = idx_v[pl.ds(g * _L, _L)]
            for k in range(_L):
                pltpu.async_copy(
                    table_hbm.at[pl.ds(lab[k], 1), :],
                    rows_v.at[pl.ds(g * _L + k, 1), :], sem)

        @pl.loop(0, b_per_w)
        def _(j):
            pltpu.make_async_copy(
                table_hbm.at[pl.ds(0, 1), :],
                rows_v.at[pl.ds(j, 1), :], sem).wait()

        pltpu.async_copy(rows_v, out_hbm.at[pl.ds(base, b_per_w)], sem2).wait()

    return _embed


def kernel(labels, train, force_drop_ids, table):
    labels32 = labels.astype(jnp.int32)
    drop32 = force_drop_ids.astype(jnp.int32)
    train_vec = jnp.full((_L,), jnp.asarray(train, dtype=jnp.int32))
    bk = _BATCH // _K
    embed = _make_embed(bk)
    parts = [
        embed(labels32[i * bk:(i + 1) * bk], train_vec,
              drop32[i * bk:(i + 1) * bk], table)
        for i in range(_K)
    ]
    return jnp.concatenate(parts, axis=0)
